# baseline (speedup 1.0000x reference)
import functools

import jax
import jax.numpy as jnp
from jax import lax
from jax.experimental import pallas as pl
from jax.experimental.pallas import tpu as pltpu
from jax.experimental.pallas import tpu_sc as plsc

_B = 4
_T = 10
_W = 5
_S = _T - _W + 1
_NW = 32
_ROWS = 224 // _NW
_NSLOT = 6


@functools.partial(
    pl.kernel,
    mesh=plsc.VectorSubcoreMesh(core_axis_name="c", subcore_axis_name="s"),
    out_type=jax.ShapeDtypeStruct((_B, _S, _W, 224, 8, 224), jnp.float32),
    compiler_params=pltpu.CompilerParams(
        use_tc_tiling_on_sc=True, skip_device_barrier=True
    ),
    scratch_types=[
        pltpu.VMEM((_NSLOT, _ROWS, 8, 224), jnp.float32),
        [pltpu.SemaphoreType.DMA] * _NSLOT,
        [pltpu.SemaphoreType.DMA] * _NSLOT,
    ],
)
def _windows_sc(in_hbm, out_hbm, buf, in_sems, out_sems):
    wid = lax.axis_index("s") * 2 + lax.axis_index("c")
    row0 = wid * _ROWS
    nf = _B * _T
    reads = [None] * _NSLOT
    pending = [[] for _ in range(_NSLOT)]

    def issue_read(fi):
        b, t = divmod(fi, _T)
        slot = fi % _NSLOT
        reads[slot] = pltpu.async_copy(
            in_hbm.at[b, t, pl.ds(row0, _ROWS)], buf.at[slot], in_sems[slot]
        )

    def prep_read(fj):
        if fj < nf:
            nslot = fj % _NSLOT
            for cp in pending[nslot]:
                cp.wait()
            pending[nslot] = []
            issue_read(fj)

    issue_read(0)
    prep_read(1)
    for fi in range(nf):
        b, t = divmod(fi, _T)
        slot = fi % _NSLOT
        prep_read(fi + 2)
        reads[slot].wait()
        for s in range(max(0, t - _W + 1), min(_S, t + 1)):
            cp = pltpu.async_copy(
                buf.at[slot],
                out_hbm.at[b, s, t - s, pl.ds(row0, _ROWS)],
                out_sems[slot],
            )
            pending[slot].append(cp)
    for slot in range(_NSLOT):
        for cp in pending[slot]:
            cp.wait()


def kernel(inputs):
    tin = jnp.transpose(inputs, (0, 1, 2, 4, 3))
    tout = _windows_sc(tin)
    return jnp.transpose(tout, (0, 1, 2, 3, 5, 4))


# speedup vs baseline: 1.0019x; 1.0019x over previous
import functools

import jax
import jax.numpy as jnp
from jax import lax
from jax.experimental import pallas as pl
from jax.experimental.pallas import tpu as pltpu
from jax.experimental.pallas import tpu_sc as plsc

_B = 4
_T = 10
_W = 5
_S = _T - _W + 1
_NW = 32
_ROWS = 224 // _NW
_NSLOT = 6


@functools.partial(
    pl.kernel,
    mesh=plsc.VectorSubcoreMesh(core_axis_name="c", subcore_axis_name="s"),
    out_type=jax.ShapeDtypeStruct((_B, _S, _W, 224, 8, 224), jnp.float32),
    compiler_params=pltpu.CompilerParams(use_tc_tiling_on_sc=True),
    scratch_types=[
        pltpu.VMEM((_NSLOT, _ROWS, 8, 224), jnp.float32),
        [pltpu.SemaphoreType.DMA] * _NSLOT,
        [pltpu.SemaphoreType.DMA] * _NSLOT,
    ],
)
def _windows_sc(in_hbm, out_hbm, buf, in_sems, out_sems):
    wid = lax.axis_index("s") * 2 + lax.axis_index("c")
    row0 = wid * _ROWS
    nf = _B * _T
    reads = [None] * _NSLOT
    pending = [[] for _ in range(_NSLOT)]

    def issue_read(fi):
        b, t = divmod(fi, _T)
        slot = fi % _NSLOT
        reads[slot] = pltpu.async_copy(
            in_hbm.at[b, t, pl.ds(row0, _ROWS)], buf.at[slot], in_sems[slot]
        )

    def prep_read(fj):
        if fj < nf:
            nslot = fj % _NSLOT
            for cp in pending[nslot]:
                cp.wait()
            pending[nslot] = []
            issue_read(fj)

    issue_read(0)
    prep_read(1)
    for fi in range(nf):
        b, t = divmod(fi, _T)
        slot = fi % _NSLOT
        prep_read(fi + 2)
        reads[slot].wait()
        for s in range(max(0, t - _W + 1), min(_S, t + 1)):
            cp = pltpu.async_copy(
                buf.at[slot],
                out_hbm.at[b, s, t - s, pl.ds(row0, _ROWS)],
                out_sems[slot],
            )
            pending[slot].append(cp)
    for slot in range(_NSLOT):
        for cp in pending[slot]:
            cp.wait()


def kernel(inputs):
    tin = jnp.transpose(inputs, (0, 1, 2, 4, 3))
    tout = _windows_sc(tin)
    return jnp.transpose(tout, (0, 1, 2, 3, 5, 4))
